# baseline (speedup 1.0000x reference)
import functools

import jax
import jax.numpy as jnp
from jax import lax
from jax.experimental import pallas as pl
from jax.experimental.pallas import tpu as pltpu
from jax.experimental.pallas import tpu_sc as plsc

_NC = 2
_NS = 16
_NW = _NC * _NS
_CH = 8


def kernel(emb, t):
    del t
    n, d = emb.shape
    rows_per_w = n // _NW
    nch = rows_per_w // _CH

    mesh = plsc.VectorSubcoreMesh(core_axis_name="c", subcore_axis_name="s")

    @functools.partial(
        pl.kernel,
        mesh=mesh,
        out_type=jax.ShapeDtypeStruct((n, d), emb.dtype),
        scratch_types=[
            pltpu.VMEM((2, _CH, d), jnp.float32),
            pltpu.SemaphoreType.DMA((2,)),
            pltpu.SemaphoreType.DMA((2,)),
        ],
    )
    def copy_k(emb_hbm, out_hbm, bufs, isems, osems):
        wid = lax.axis_index("s") * _NC + lax.axis_index("c")
        base = wid * rows_per_w

        def in_copy(i):
            b = i % 2
            return pltpu.make_async_copy(
                emb_hbm.at[pl.ds(base + i * _CH, _CH)], bufs.at[b], isems.at[b]
            )

        def out_copy(i):
            b = i % 2
            return pltpu.make_async_copy(
                bufs.at[b], out_hbm.at[pl.ds(base + i * _CH, _CH)], osems.at[b]
            )

        waited = set()

        def start_load(j):
            k = j - 2
            if k >= 0 and k not in waited:
                out_copy(k).wait()
                waited.add(k)
            in_copy(j).start()

        start_load(0)
        for i in range(nch):
            if i + 1 < nch:
                start_load(i + 1)
            in_copy(i).wait()
            out_copy(i).start()
        for k in range(nch):
            if k not in waited:
                out_copy(k).wait()

    return copy_k(emb)


# speedup vs baseline: 1.3838x; 1.3838x over previous
import jax
import jax.numpy as jnp
from jax.experimental import pallas as pl
from jax.experimental.pallas import tpu as pltpu

_SIZES = [16, 16, 32, 64, 128, 256] + [512] * 14 + [256, 128, 64, 32, 16, 16]
_POOL = 3840
_LOOKAHEAD = 7


def _plan():
    hbm_off, pool_off = [], []
    h = 0
    c = 0
    for sz in _SIZES:
        if c + sz > _POOL:
            c = 0
        hbm_off.append(h)
        pool_off.append(c)
        h += sz
        c += sz
    assert h == 8192
    return hbm_off, pool_off


def _ring_body(emb_hbm, out_hbm, pool, in_sems, out_sems):
    nch = len(_SIZES)
    hbm_off, pool_off = _plan()

    def in_copy(i):
        return pltpu.make_async_copy(
            emb_hbm.at[pl.ds(hbm_off[i], _SIZES[i])],
            pool.at[pl.ds(pool_off[i], _SIZES[i])],
            in_sems.at[i],
        )

    def out_copy(i):
        return pltpu.make_async_copy(
            pool.at[pl.ds(pool_off[i], _SIZES[i])],
            out_hbm.at[pl.ds(hbm_off[i], _SIZES[i])],
            out_sems.at[i],
        )

    waited = set()

    def start_load(j):
        lo, hi = pool_off[j], pool_off[j] + _SIZES[j]
        for k in range(j):
            if k in waited:
                continue
            klo, khi = pool_off[k], pool_off[k] + _SIZES[k]
            if klo < hi and lo < khi:
                out_copy(k).wait()
                waited.add(k)
        in_copy(j).start()

    for j in range(min(_LOOKAHEAD, nch)):
        start_load(j)
    for i in range(nch):
        in_copy(i).wait()
        out_copy(i).start()
        j = i + _LOOKAHEAD
        if j < nch:
            start_load(j)
    for k in range(nch):
        if k not in waited:
            out_copy(k).wait()


def kernel(emb, t):
    del t
    n, d = emb.shape
    nch = len(_SIZES)
    return pl.pallas_call(
        _ring_body,
        in_specs=[pl.BlockSpec(memory_space=pl.ANY)],
        out_specs=pl.BlockSpec(memory_space=pl.ANY),
        out_shape=jax.ShapeDtypeStruct((n, d), emb.dtype),
        scratch_shapes=[
            pltpu.VMEM((_POOL, d), jnp.float32),
            pltpu.SemaphoreType.DMA((nch,)),
            pltpu.SemaphoreType.DMA((nch,)),
        ],
        compiler_params=pltpu.CompilerParams(skip_device_barrier=True, vmem_limit_bytes=63 * 1024 * 1024),
    )(emb)
